# baseline (speedup 1.0000x reference)
import functools

import jax
import jax.numpy as jnp
from jax import lax
from jax.experimental import pallas as pl
from jax.experimental.pallas import tpu as pltpu
from jax.experimental.pallas import tpu_sc as plsc

N = 10000
E = 320000
NG = 8
DH = 128
NP = 10240
E_TOT = 332800
NC = 2
NS = 16
NW = NC * NS

EDGES_PER_W = E_TOT // NW
K_AGG = 104
NWIN_AGG = EDGES_PER_W // K_AGG
KH = 48
KH2 = K_AGG - KH
K_NRM = 1040
NWIN_NRM = EDGES_PER_W // K_NRM
EDGES_PER_T = E_TOT // NS
K_DEG = 2600
NWIN_DEG = EDGES_PER_T // K_DEG

_MESH = plsc.VectorSubcoreMesh(core_axis_name="c", subcore_axis_name="s")


def _newton_rsqrt(x):
    xb = lax.bitcast_convert_type(x, jnp.int32)
    y = lax.bitcast_convert_type(jnp.int32(0x5F3759DF) - (xb >> 1),
                                 jnp.float32)
    for _ in range(4):
        y = y * (1.5 - 0.5 * x * y * y)
    return y


def _sc_norm_body(src_hbm, dst_hbm, ew_hbm, batch_hbm, zeros_hbm,
                  norm_hbm, qpart_hbm,
                  deg_acc, q_acc, dis_sh,
                  dstb_a, ewb_a, dbuf, disb, dis_full, batch_full,
                  srcb, dstb, ewb, normb, qidxb):
    cid = lax.axis_index("c")
    tid = lax.axis_index("s")
    wid = tid * NC + cid

    @pl.when(tid == 0)
    def _():
        pltpu.sync_copy(zeros_hbm.at[pl.ds(0, NP)], deg_acc)
        pltpu.sync_copy(zeros_hbm, q_acc)

    plsc.subcore_barrier()

    def deg_win(w, carry):
        base = pl.multiple_of(tid * EDGES_PER_T + w * K_DEG, 8)
        pltpu.sync_copy(dst_hbm.at[pl.ds(base, K_DEG)], dstb_a)
        pltpu.sync_copy(ew_hbm.at[pl.ds(base, K_DEG)], ewb_a)
        pltpu.sync_copy(ewb_a, deg_acc.at[dstb_a], add=True)
        return carry

    lax.fori_loop(0, NWIN_DEG, deg_win, 0)
    plsc.subcore_barrier()

    rb = tid * (NP // NS)
    pltpu.sync_copy(deg_acc.at[pl.ds(rb, NP // NS)], dbuf)
    for i in range(NP // NS // 16):
        sl = pl.ds(i * 16, 16)
        disb[sl] = _newton_rsqrt(dbuf[sl])
    pltpu.sync_copy(disb, dis_sh.at[pl.ds(rb, NP // NS)])
    plsc.subcore_barrier()
    pltpu.sync_copy(dis_sh, dis_full)
    pltpu.sync_copy(batch_hbm, batch_full.at[pl.ds(0, N)])

    def nrm_win(w, carry):
        base = pl.multiple_of(wid * EDGES_PER_W + w * K_NRM, 8)
        pltpu.sync_copy(src_hbm.at[pl.ds(base, K_NRM)], srcb)
        pltpu.sync_copy(dst_hbm.at[pl.ds(base, K_NRM)], dstb)
        pltpu.sync_copy(ew_hbm.at[pl.ds(base, K_NRM)], ewb)
        for g in range(K_NRM // 16):
            sl = pl.ds(g * 16, 16)
            s16 = srcb[sl]
            d16 = dstb[sl]
            gs = plsc.load_gather(dis_full, [s16])
            gd = plsc.load_gather(dis_full, [d16])
            normb[sl] = gs * ewb[sl] * gd
            bg = plsc.load_gather(batch_full, [d16])
            qidxb[sl] = s16 * NG + bg
        pltpu.sync_copy(normb, norm_hbm.at[pl.ds(base, K_NRM)])
        pltpu.sync_copy(normb, q_acc.at[qidxb], add=True)
        return carry

    lax.fori_loop(0, NWIN_NRM, nrm_win, 0)
    plsc.subcore_barrier()

    qb = pl.multiple_of(tid * 5120, 128)

    @pl.when(tid < NS - 1)
    def _():
        pltpu.sync_copy(q_acc.at[pl.ds(qb, 5120)],
                        qpart_hbm.at[cid, 0, pl.ds(qb, 5120)])

    @pl.when(tid == NS - 1)
    def _():
        pltpu.sync_copy(q_acc.at[pl.ds(qb, 3200)],
                        qpart_hbm.at[cid, 0, pl.ds(qb, 3200)])


_sc_norm = pl.kernel(
    _sc_norm_body,
    out_type=(
        jax.ShapeDtypeStruct((E_TOT,), jnp.float32),
        jax.ShapeDtypeStruct((NC, 1, NG * N), jnp.float32),
    ),
    mesh=_MESH,
    scratch_types=[
        pltpu.VMEM_SHARED((NP,), jnp.float32),
        pltpu.VMEM_SHARED((NG * N,), jnp.float32),
        pltpu.VMEM_SHARED((NP,), jnp.float32),
        pltpu.VMEM((K_DEG,), jnp.int32),
        pltpu.VMEM((K_DEG,), jnp.float32),
        pltpu.VMEM((NP // NS,), jnp.float32),
        pltpu.VMEM((NP // NS,), jnp.float32),
        pltpu.VMEM((NP,), jnp.float32),
        pltpu.VMEM((NP,), jnp.int32),
        pltpu.VMEM((K_NRM,), jnp.int32),
        pltpu.VMEM((K_NRM,), jnp.int32),
        pltpu.VMEM((K_NRM,), jnp.float32),
        pltpu.VMEM((K_NRM,), jnp.float32),
        pltpu.VMEM((K_NRM,), jnp.int32),
    ],
    compiler_params=pltpu.CompilerParams(needs_layout_passes=False),
    name="sc_gcn_norm",
)


def _sc_agg_body(h_hbm, src_hbm, dst_hbm, norm_hbm, zeros_hbm,
                 parts_hbm,
                 acc, srcb0, dstb0, nrmb0, rows0,
                 srcb1, dstb1, nrmb1, rows1, semi, semg, sems):
    cid = lax.axis_index("c")
    tid = lax.axis_index("s")
    wid = tid * NC + cid

    @pl.when(tid == 0)
    def _():
        pltpu.sync_copy(zeros_hbm, acc)

    plsc.subcore_barrier()

    eb = wid * EDGES_PER_W
    bufs0 = (rows0, srcb0, dstb0, nrmb0)
    bufs1 = (rows1, srcb1, dstb1, nrmb1)

    def stage(w, bufs):
        _, sb, db, nb = bufs
        base = pl.multiple_of(eb + w * K_AGG, 8)
        pltpu.async_copy(src_hbm.at[pl.ds(base, K_AGG)], sb, semi)
        pltpu.async_copy(dst_hbm.at[pl.ds(base, K_AGG)], db, semi)
        pltpu.async_copy(norm_hbm.at[pl.ds(base, K_AGG)], nb, semi)

    def wait_stage(bufs):
        _, sb, db, nb = bufs
        s0 = pl.ds(0, K_AGG)
        pltpu.make_async_copy(src_hbm.at[s0], sb, semi).wait()
        pltpu.make_async_copy(dst_hbm.at[s0], db, semi).wait()
        pltpu.make_async_copy(norm_hbm.at[s0], nb, semi).wait()

    pltpu.sync_copy(src_hbm.at[pl.ds(eb, K_AGG)], srcb0)
    pltpu.sync_copy(dst_hbm.at[pl.ds(eb, K_AGG)], dstb0)
    pltpu.sync_copy(norm_hbm.at[pl.ds(eb, K_AGG)], nrmb0)
    pltpu.async_copy(h_hbm.at[srcb0.at[pl.ds(0, KH)]],
                     rows0.at[pl.ds(0, KH)], semg)
    pltpu.async_copy(h_hbm.at[srcb0.at[pl.ds(KH, KH2)]],
                     rows0.at[pl.ds(KH, KH2)], semg)

    def process(w, cur, nxt_bufs, have_next):
        rw, _, dw, nw = cur
        rn, sn, dn_, _ = nxt_bufs
        nxt = w + 1

        @pl.when(have_next)
        def _():
            stage(nxt, nxt_bufs)

        pltpu.make_async_copy(h_hbm.at[cur[1].at[pl.ds(0, KH)]],
                              rw.at[pl.ds(0, KH)], semg).wait()
        pltpu.make_async_copy(h_hbm.at[cur[1].at[pl.ds(KH, KH2)]],
                              rw.at[pl.ds(KH, KH2)], semg).wait()

        @pl.when(have_next)
        def _():
            wait_stage(nxt_bufs)

            @pl.when(w >= 1)
            def _():
                pltpu.make_async_copy(rn, acc.at[dn_], sems).wait()

            pltpu.async_copy(h_hbm.at[sn.at[pl.ds(0, KH)]],
                             rn.at[pl.ds(0, KH)], semg)
            pltpu.async_copy(h_hbm.at[sn.at[pl.ds(KH, KH2)]],
                             rn.at[pl.ds(KH, KH2)], semg)

        @plsc.parallel_loop(0, K_AGG, step=1, unroll=8)
        def _(e):
            nv = plsc.load_gather(nw, [jnp.zeros((16,), jnp.int32) + e])
            for c in range(DH // 16):
                sl = pl.ds(c * 16, 16)
                rw[e, sl] = rw[e, sl] * nv

        pltpu.async_copy(rw, acc.at[dw], sems, add=True)

    def win_pair(p, carry):
        w0 = p * 2
        process(w0, bufs0, bufs1, w0 + 1 < NWIN_AGG)
        process(w0 + 1, bufs1, bufs0, w0 + 2 < NWIN_AGG)
        return carry

    lax.fori_loop(0, NWIN_AGG // 2, win_pair, 0)
    pltpu.make_async_copy(rows0, acc.at[dstb0], sems).wait()
    pltpu.make_async_copy(rows1, acc.at[dstb1], sems).wait()
    plsc.subcore_barrier()

    rb = pl.multiple_of(tid * 624, 8)

    @pl.when(tid < NS - 1)
    def _():
        pltpu.sync_copy(acc.at[pl.ds(rb, 624)],
                        parts_hbm.at[cid, pl.ds(rb, 624)])

    @pl.when(tid == NS - 1)
    def _():
        pltpu.sync_copy(acc.at[pl.ds(rb, 640)],
                        parts_hbm.at[cid, pl.ds(rb, 640)])


_sc_agg = pl.kernel(
    _sc_agg_body,
    out_type=jax.ShapeDtypeStruct((NC, N, DH), jnp.float32),
    mesh=_MESH,
    scratch_types=[
        pltpu.VMEM_SHARED((N, DH), jnp.float32),
        pltpu.VMEM((K_AGG,), jnp.int32),
        pltpu.VMEM((K_AGG,), jnp.int32),
        pltpu.VMEM((K_AGG,), jnp.float32),
        pltpu.VMEM((K_AGG, DH), jnp.float32),
        pltpu.VMEM((K_AGG,), jnp.int32),
        pltpu.VMEM((K_AGG,), jnp.int32),
        pltpu.VMEM((K_AGG,), jnp.float32),
        pltpu.VMEM((K_AGG, DH), jnp.float32),
        pltpu.SemaphoreType.DMA,
        pltpu.SemaphoreType.DMA,
        pltpu.SemaphoreType.DMA,
    ],
    compiler_params=pltpu.CompilerParams(needs_layout_passes=False),
    name="sc_gcn_agg",
)


RB = 1000
NB = N // RB


def _mm_body(x_ref, w_ref, o_ref):
    o_ref[...] = jnp.dot(x_ref[...], w_ref[...],
                         preferred_element_type=jnp.float32)


_tc_mm = pl.pallas_call(
    _mm_body,
    grid=(NB,),
    in_specs=[
        pl.BlockSpec((RB, DH), lambda i: (i, 0)),
        pl.BlockSpec((DH, DH), lambda i: (0, 0)),
    ],
    out_specs=pl.BlockSpec((RB, DH), lambda i: (i, 0)),
    out_shape=jax.ShapeDtypeStruct((N, DH), jnp.float32),
)


def _relu_mm_body(p_ref, b_ref, w_ref, o_ref):
    h = jnp.maximum(p_ref[0] + p_ref[1] + b_ref[...], 0.0)
    o_ref[...] = jnp.dot(h, w_ref[...], preferred_element_type=jnp.float32)


_tc_relu_mm = pl.pallas_call(
    _relu_mm_body,
    grid=(NB,),
    in_specs=[
        pl.BlockSpec((NC, RB, DH), lambda i: (0, i, 0)),
        pl.BlockSpec((1, DH), lambda i: (0, 0)),
        pl.BlockSpec((DH, DH), lambda i: (0, 0)),
    ],
    out_specs=pl.BlockSpec((RB, DH), lambda i: (i, 0)),
    out_shape=jax.ShapeDtypeStruct((N, DH), jnp.float32),
)


def _head_body(p_ref, b2_ref, qp_ref, batch_ref, w3_ref, b3_ref,
               g_ref, be_ref, p1w_ref, p1b_ref, p2w_ref, p2b_ref,
               o_ref, z_acc, c_acc):
    i = pl.program_id(0)

    @pl.when(i == 0)
    def _():
        z_acc[...] = jnp.zeros_like(z_acc)
        c_acc[...] = jnp.zeros_like(c_acc)

    h2 = jnp.maximum(p_ref[0] + p_ref[1] + b2_ref[...], 0.0)
    hm = jnp.dot(h2, w3_ref[...],
                 preferred_element_type=jnp.float32)
    qs = qp_ref[0] + qp_ref[1]
    dn = (((0,), (0,)), ((), ()))
    z_acc[...] += lax.dot_general(qs, hm, dimension_numbers=dn,
                                  preferred_element_type=jnp.float32,
                                  precision=lax.Precision.HIGHEST)
    m = (batch_ref[...] ==
         lax.broadcasted_iota(jnp.int32, (RB, NG), 1)).astype(jnp.float32)
    cnt = lax.dot_general(m, jnp.ones((RB, 1), jnp.float32),
                          dimension_numbers=dn,
                          preferred_element_type=jnp.float32,
                          precision=lax.Precision.HIGHEST)
    c_acc[...] += jnp.broadcast_to(cnt, (NG, 200))

    @pl.when(i == NB - 1)
    def _():
        counts = jnp.maximum(c_acc[...], 1.0)
        y = z_acc[...] / counts + b3_ref[...]
        mu = jnp.mean(y, axis=-1, keepdims=True)
        var = jnp.mean((y - mu) ** 2, axis=-1, keepdims=True)
        yn = (y - mu) * lax.rsqrt(var + 1e-5) * g_ref[...] + be_ref[...]
        t = jnp.maximum(
            jnp.dot(yn, p1w_ref[...],
                    preferred_element_type=jnp.float32) + p1b_ref[...], 0.0)
        o_ref[...] = jnp.dot(t, p2w_ref[...],
                             preferred_element_type=jnp.float32) + p2b_ref[...]


_tc_head = pl.pallas_call(
    _head_body,
    grid=(NB,),
    in_specs=[
        pl.BlockSpec((NC, RB, DH), lambda i: (0, i, 0)),
        pl.BlockSpec((1, DH), lambda i: (0, 0)),
        pl.BlockSpec((NC, RB, NG), lambda i: (0, i, 0)),
        pl.BlockSpec((RB, 1), lambda i: (i, 0)),
        pl.BlockSpec((DH, 200), lambda i: (0, 0)),
        pl.BlockSpec((1, 200), lambda i: (0, 0)),
        pl.BlockSpec((1, 200), lambda i: (0, 0)),
        pl.BlockSpec((1, 200), lambda i: (0, 0)),
        pl.BlockSpec((200, DH), lambda i: (0, 0)),
        pl.BlockSpec((1, DH), lambda i: (0, 0)),
        pl.BlockSpec((DH, 4), lambda i: (0, 0)),
        pl.BlockSpec((1, 4), lambda i: (0, 0)),
    ],
    out_specs=pl.BlockSpec((NG, 4), lambda i: (0, 0)),
    out_shape=jax.ShapeDtypeStruct((NG, 4), jnp.float32),
    scratch_shapes=[
        pltpu.VMEM((NG, 200), jnp.float32),
        pltpu.VMEM((NG, 200), jnp.float32),
    ],
)


def kernel(x, edge_index, edge_attr, batch, W1, b1, W2, b2, W3, b3,
           ln_gamma, ln_beta, P1_W, P1_b, P2_W, P2_b):
    src = edge_index[0]
    dst = edge_index[1]
    iota = jnp.arange(N, dtype=jnp.int32)
    padn = E_TOT - E - N
    zpad_i = jnp.zeros((padn,), jnp.int32)
    src_e = jnp.concatenate([src, iota, zpad_i])
    dst_e = jnp.concatenate([dst, iota, zpad_i])
    ew_e = jnp.concatenate([edge_attr, jnp.ones((N,), jnp.float32),
                            jnp.zeros((padn,), jnp.float32)])
    zeros_q = jnp.zeros((NG * N,), jnp.float32)
    zeros_h = jnp.zeros((N, DH), jnp.float32)

    norm, qpart = _sc_norm(src_e, dst_e, ew_e, batch, zeros_q)

    h1a = _tc_mm(x, W1)
    p1 = _sc_agg(h1a, src_e, dst_e, norm, zeros_h)
    h2a = _tc_relu_mm(p1, b1.reshape(1, DH), W2)
    p2 = _sc_agg(h2a, src_e, dst_e, norm, zeros_h)

    return _tc_head(
        p2, b2.reshape(1, DH), qpart.reshape(NC, N, NG),
        batch.reshape(N, 1), W3, b3.reshape(1, 200),
        ln_gamma.reshape(1, 200), ln_beta.reshape(1, 200),
        P1_W, P1_b.reshape(1, DH), P2_W, P2_b.reshape(1, 4))


# speedup vs baseline: 1.0914x; 1.0914x over previous
import functools

import jax
import jax.numpy as jnp
from jax import lax
from jax.experimental import pallas as pl
from jax.experimental.pallas import tpu as pltpu
from jax.experimental.pallas import tpu_sc as plsc

N = 10000
E = 320000
NG = 8
DH = 128
NP = 10240
E_TOT = 331776
NC = 2
NS = 16
NW = NC * NS

EDGES_PER_W = E_TOT // NW
K_AGG = 128
NWIN_AGG = EDGES_PER_W // K_AGG
K_NRM = 1296
NWIN_NRM = EDGES_PER_W // K_NRM
EDGES_PER_T = E_TOT // NS
K_DEG = 2592
NWIN_DEG = EDGES_PER_T // K_DEG

_MESH = plsc.VectorSubcoreMesh(core_axis_name="c", subcore_axis_name="s")


def _newton_rsqrt(x):
    xb = lax.bitcast_convert_type(x, jnp.int32)
    y = lax.bitcast_convert_type(jnp.int32(0x5F3759DF) - (xb >> 1),
                                 jnp.float32)
    for _ in range(4):
        y = y * (1.5 - 0.5 * x * y * y)
    return y


def _sc_norm_body(src_hbm, dst_hbm, ew_hbm, batch_hbm, zeros_hbm,
                  norm_hbm, qpart_hbm,
                  deg_acc, q_acc, dis_sh,
                  dstb_a, ewb_a, dbuf, disb, dis_full, batch_full,
                  srcb, dstb, ewb, normb, qidxb):
    cid = lax.axis_index("c")
    tid = lax.axis_index("s")
    wid = tid * NC + cid

    @pl.when(tid == 0)
    def _():
        pltpu.sync_copy(zeros_hbm.at[pl.ds(0, NP)], deg_acc)
        pltpu.sync_copy(zeros_hbm, q_acc)

    plsc.subcore_barrier()

    def deg_win(w, carry):
        base = pl.multiple_of(tid * EDGES_PER_T + w * K_DEG, 8)
        pltpu.sync_copy(dst_hbm.at[pl.ds(base, K_DEG)], dstb_a)
        pltpu.sync_copy(ew_hbm.at[pl.ds(base, K_DEG)], ewb_a)
        pltpu.sync_copy(ewb_a, deg_acc.at[dstb_a], add=True)
        return carry

    lax.fori_loop(0, NWIN_DEG, deg_win, 0)
    plsc.subcore_barrier()

    rb = tid * (NP // NS)
    pltpu.sync_copy(deg_acc.at[pl.ds(rb, NP // NS)], dbuf)
    for i in range(NP // NS // 16):
        sl = pl.ds(i * 16, 16)
        disb[sl] = _newton_rsqrt(dbuf[sl])
    pltpu.sync_copy(disb, dis_sh.at[pl.ds(rb, NP // NS)])
    plsc.subcore_barrier()
    pltpu.sync_copy(dis_sh, dis_full)
    pltpu.sync_copy(batch_hbm, batch_full.at[pl.ds(0, N)])

    def nrm_win(w, carry):
        base = pl.multiple_of(wid * EDGES_PER_W + w * K_NRM, 8)
        pltpu.sync_copy(src_hbm.at[pl.ds(base, K_NRM)], srcb)
        pltpu.sync_copy(dst_hbm.at[pl.ds(base, K_NRM)], dstb)
        pltpu.sync_copy(ew_hbm.at[pl.ds(base, K_NRM)], ewb)
        for g in range(K_NRM // 16):
            sl = pl.ds(g * 16, 16)
            s16 = srcb[sl]
            d16 = dstb[sl]
            gs = plsc.load_gather(dis_full, [s16])
            gd = plsc.load_gather(dis_full, [d16])
            normb[sl] = gs * ewb[sl] * gd
            bg = plsc.load_gather(batch_full, [d16])
            qidxb[sl] = s16 * NG + bg
        pltpu.sync_copy(normb, norm_hbm.at[pl.ds(base, K_NRM)])
        pltpu.sync_copy(normb, q_acc.at[qidxb], add=True)
        return carry

    lax.fori_loop(0, NWIN_NRM, nrm_win, 0)
    plsc.subcore_barrier()

    qb = pl.multiple_of(tid * 5120, 128)

    @pl.when(tid < NS - 1)
    def _():
        pltpu.sync_copy(q_acc.at[pl.ds(qb, 5120)],
                        qpart_hbm.at[cid, 0, pl.ds(qb, 5120)])

    @pl.when(tid == NS - 1)
    def _():
        pltpu.sync_copy(q_acc.at[pl.ds(qb, 3200)],
                        qpart_hbm.at[cid, 0, pl.ds(qb, 3200)])


_sc_norm = pl.kernel(
    _sc_norm_body,
    out_type=(
        jax.ShapeDtypeStruct((E_TOT,), jnp.float32),
        jax.ShapeDtypeStruct((NC, 1, NG * N), jnp.float32),
    ),
    mesh=_MESH,
    scratch_types=[
        pltpu.VMEM_SHARED((NP,), jnp.float32),
        pltpu.VMEM_SHARED((NG * N,), jnp.float32),
        pltpu.VMEM_SHARED((NP,), jnp.float32),
        pltpu.VMEM((K_DEG,), jnp.int32),
        pltpu.VMEM((K_DEG,), jnp.float32),
        pltpu.VMEM((NP // NS,), jnp.float32),
        pltpu.VMEM((NP // NS,), jnp.float32),
        pltpu.VMEM((NP,), jnp.float32),
        pltpu.VMEM((NP,), jnp.int32),
        pltpu.VMEM((K_NRM,), jnp.int32),
        pltpu.VMEM((K_NRM,), jnp.int32),
        pltpu.VMEM((K_NRM,), jnp.float32),
        pltpu.VMEM((K_NRM,), jnp.float32),
        pltpu.VMEM((K_NRM,), jnp.int32),
    ],
    compiler_params=pltpu.CompilerParams(needs_layout_passes=False),
    name="sc_gcn_norm",
)


def _sc_agg_body(h_hbm, ed_hbm, dst_hbm, zeros_hbm,
                 parts_hbm,
                 acc, ep0, dstb0, rows0, ep1, dstb1, rows1,
                 semi, semg, sems):
    cid = lax.axis_index("c")
    tid = lax.axis_index("s")
    wid = tid * NC + cid

    @pl.when(tid == 0)
    def _():
        pltpu.sync_copy(zeros_hbm, acc)

    plsc.subcore_barrier()

    eb = wid * EDGES_PER_W

    def stage(w, ep, db):
        base = pl.multiple_of(eb + w * K_AGG, 128)
        pltpu.async_copy(ed_hbm.at[:, pl.ds(base, K_AGG)], ep, semi)
        pltpu.async_copy(dst_hbm.at[pl.ds(base, K_AGG)], db, semi)

    pltpu.sync_copy(ed_hbm.at[:, pl.ds(eb, K_AGG)], ep0)
    pltpu.sync_copy(dst_hbm.at[pl.ds(eb, K_AGG)], dstb0)
    pltpu.async_copy(h_hbm.at[ep0.at[0]], rows0, semg)

    def process(w, ep, db, rw, epn, dbn, rn, have_next):
        nxt = w + 1

        @pl.when(w >= 1)
        def _():
            pltpu.make_async_copy(rn, acc.at[dbn], sems).wait()

        @pl.when(have_next)
        def _():
            stage(nxt, epn, dbn)

        pltpu.make_async_copy(h_hbm.at[ep.at[0]], rw, semg).wait()

        @pl.when(have_next)
        def _():
            pltpu.make_async_copy(
                ed_hbm.at[:, pl.ds(0, K_AGG)], epn, semi).wait()
            pltpu.make_async_copy(
                dst_hbm.at[pl.ds(0, K_AGG)], dbn, semi).wait()
            pltpu.async_copy(h_hbm.at[epn.at[0]], rn, semg)

        @plsc.parallel_loop(0, K_AGG, step=1, unroll=8)
        def _(e):
            nv = lax.bitcast_convert_type(
                plsc.load_gather(ep.at[1],
                                 [jnp.zeros((16,), jnp.int32) + e]),
                jnp.float32)
            for c in range(DH // 16):
                sl = pl.ds(c * 16, 16)
                rw[e, sl] = rw[e, sl] * nv

        pltpu.async_copy(rw, acc.at[db], sems, add=True)

    def win_pair(p, carry):
        w0 = p * 2
        process(w0, ep0, dstb0, rows0, ep1, dstb1, rows1,
                w0 + 1 < NWIN_AGG)
        process(w0 + 1, ep1, dstb1, rows1, ep0, dstb0, rows0,
                w0 + 2 < NWIN_AGG)
        return carry

    lax.fori_loop(0, NWIN_AGG // 2, win_pair, 0)
    process(NWIN_AGG - 1, ep0, dstb0, rows0, ep1, dstb1, rows1, False)
    pltpu.make_async_copy(rows0, acc.at[dstb0], sems).wait()
    plsc.subcore_barrier()

    rb = pl.multiple_of(tid * 624, 8)

    @pl.when(tid < NS - 1)
    def _():
        pltpu.sync_copy(acc.at[pl.ds(rb, 624)],
                        parts_hbm.at[cid, pl.ds(rb, 624)])

    @pl.when(tid == NS - 1)
    def _():
        pltpu.sync_copy(acc.at[pl.ds(rb, 640)],
                        parts_hbm.at[cid, pl.ds(rb, 640)])


_sc_agg = pl.kernel(
    _sc_agg_body,
    out_type=jax.ShapeDtypeStruct((NC, N, DH), jnp.float32),
    mesh=_MESH,
    scratch_types=[
        pltpu.VMEM_SHARED((N, DH), jnp.float32),
        pltpu.VMEM((2, K_AGG), jnp.int32),
        pltpu.VMEM((K_AGG,), jnp.int32),
        pltpu.VMEM((K_AGG, DH), jnp.float32),
        pltpu.VMEM((2, K_AGG), jnp.int32),
        pltpu.VMEM((K_AGG,), jnp.int32),
        pltpu.VMEM((K_AGG, DH), jnp.float32),
        pltpu.SemaphoreType.DMA,
        pltpu.SemaphoreType.DMA,
        pltpu.SemaphoreType.DMA,
    ],
    compiler_params=pltpu.CompilerParams(needs_layout_passes=False),
    name="sc_gcn_agg",
)


RB = 1000
NB = N // RB


def _mm_body(x_ref, w_ref, o_ref):
    o_ref[...] = jnp.dot(x_ref[...], w_ref[...],
                         preferred_element_type=jnp.float32)


_tc_mm = pl.pallas_call(
    _mm_body,
    grid=(NB,),
    in_specs=[
        pl.BlockSpec((RB, DH), lambda i: (i, 0)),
        pl.BlockSpec((DH, DH), lambda i: (0, 0)),
    ],
    out_specs=pl.BlockSpec((RB, DH), lambda i: (i, 0)),
    out_shape=jax.ShapeDtypeStruct((N, DH), jnp.float32),
)


def _relu_mm_body(p_ref, b_ref, w_ref, o_ref):
    h = jnp.maximum(p_ref[0] + p_ref[1] + b_ref[...], 0.0)
    o_ref[...] = jnp.dot(h, w_ref[...], preferred_element_type=jnp.float32)


_tc_relu_mm = pl.pallas_call(
    _relu_mm_body,
    grid=(NB,),
    in_specs=[
        pl.BlockSpec((NC, RB, DH), lambda i: (0, i, 0)),
        pl.BlockSpec((1, DH), lambda i: (0, 0)),
        pl.BlockSpec((DH, DH), lambda i: (0, 0)),
    ],
    out_specs=pl.BlockSpec((RB, DH), lambda i: (i, 0)),
    out_shape=jax.ShapeDtypeStruct((N, DH), jnp.float32),
)


def _head_body(p_ref, b2_ref, qp_ref, batch_ref, w3_ref, b3_ref,
               g_ref, be_ref, p1w_ref, p1b_ref, p2w_ref, p2b_ref,
               o_ref, z_acc, c_acc):
    i = pl.program_id(0)

    @pl.when(i == 0)
    def _():
        z_acc[...] = jnp.zeros_like(z_acc)
        c_acc[...] = jnp.zeros_like(c_acc)

    h2 = jnp.maximum(p_ref[0] + p_ref[1] + b2_ref[...], 0.0)
    hm = jnp.dot(h2, w3_ref[...],
                 preferred_element_type=jnp.float32)
    qs = qp_ref[0] + qp_ref[1]
    dn = (((0,), (0,)), ((), ()))
    z_acc[...] += lax.dot_general(qs, hm, dimension_numbers=dn,
                                  preferred_element_type=jnp.float32,
                                  precision=lax.Precision.HIGHEST)
    m = (batch_ref[...] ==
         lax.broadcasted_iota(jnp.int32, (RB, NG), 1)).astype(jnp.float32)
    cnt = lax.dot_general(m, jnp.ones((RB, 1), jnp.float32),
                          dimension_numbers=dn,
                          preferred_element_type=jnp.float32,
                          precision=lax.Precision.HIGHEST)
    c_acc[...] += jnp.broadcast_to(cnt, (NG, 200))

    @pl.when(i == NB - 1)
    def _():
        counts = jnp.maximum(c_acc[...], 1.0)
        y = z_acc[...] / counts + b3_ref[...]
        mu = jnp.mean(y, axis=-1, keepdims=True)
        var = jnp.mean((y - mu) ** 2, axis=-1, keepdims=True)
        yn = (y - mu) * lax.rsqrt(var + 1e-5) * g_ref[...] + be_ref[...]
        t = jnp.maximum(
            jnp.dot(yn, p1w_ref[...],
                    preferred_element_type=jnp.float32) + p1b_ref[...], 0.0)
        o_ref[...] = jnp.dot(t, p2w_ref[...],
                             preferred_element_type=jnp.float32) + p2b_ref[...]


_tc_head = pl.pallas_call(
    _head_body,
    grid=(NB,),
    in_specs=[
        pl.BlockSpec((NC, RB, DH), lambda i: (0, i, 0)),
        pl.BlockSpec((1, DH), lambda i: (0, 0)),
        pl.BlockSpec((NC, RB, NG), lambda i: (0, i, 0)),
        pl.BlockSpec((RB, 1), lambda i: (i, 0)),
        pl.BlockSpec((DH, 200), lambda i: (0, 0)),
        pl.BlockSpec((1, 200), lambda i: (0, 0)),
        pl.BlockSpec((1, 200), lambda i: (0, 0)),
        pl.BlockSpec((1, 200), lambda i: (0, 0)),
        pl.BlockSpec((200, DH), lambda i: (0, 0)),
        pl.BlockSpec((1, DH), lambda i: (0, 0)),
        pl.BlockSpec((DH, 4), lambda i: (0, 0)),
        pl.BlockSpec((1, 4), lambda i: (0, 0)),
    ],
    out_specs=pl.BlockSpec((NG, 4), lambda i: (0, 0)),
    out_shape=jax.ShapeDtypeStruct((NG, 4), jnp.float32),
    scratch_shapes=[
        pltpu.VMEM((NG, 200), jnp.float32),
        pltpu.VMEM((NG, 200), jnp.float32),
    ],
)


def kernel(x, edge_index, edge_attr, batch, W1, b1, W2, b2, W3, b3,
           ln_gamma, ln_beta, P1_W, P1_b, P2_W, P2_b):
    src = edge_index[0]
    dst = edge_index[1]
    iota = jnp.arange(N, dtype=jnp.int32)
    padn = E_TOT - E - N
    zpad_i = jnp.zeros((padn,), jnp.int32)
    src_e = jnp.concatenate([src, iota, zpad_i])
    dst_e = jnp.concatenate([dst, iota, zpad_i])
    ew_e = jnp.concatenate([edge_attr, jnp.ones((N,), jnp.float32),
                            jnp.zeros((padn,), jnp.float32)])
    zeros_q = jnp.zeros((NG * N,), jnp.float32)
    zeros_h = jnp.zeros((N, DH), jnp.float32)

    norm, qpart = _sc_norm(src_e, dst_e, ew_e, batch, zeros_q)
    ed = jnp.stack([src_e, lax.bitcast_convert_type(norm, jnp.int32)])

    h1a = _tc_mm(x, W1)
    p1 = _sc_agg(h1a, ed, dst_e, zeros_h)
    h2a = _tc_relu_mm(p1, b1.reshape(1, DH), W2)
    p2 = _sc_agg(h2a, ed, dst_e, zeros_h)

    return _tc_head(
        p2, b2.reshape(1, DH), qpart.reshape(NC, N, NG),
        batch.reshape(N, 1), W3, b3.reshape(1, 200),
        ln_gamma.reshape(1, 200), ln_beta.reshape(1, 200),
        P1_W, P1_b.reshape(1, DH), P2_W, P2_b.reshape(1, 4))


# speedup vs baseline: 1.0922x; 1.0007x over previous
import jax
import jax.numpy as jnp
from jax import lax
from jax.experimental import pallas as pl
from jax.experimental.pallas import tpu as pltpu
from jax.experimental.pallas import tpu_sc as plsc

N = 10000
E = 320000
NG = 8
DH = 128
NP = 10240
E_TOT = 331776
NC = 2
NS = 16
NW = NC * NS

EDGES_PER_W = E_TOT // NW
K_AGG = 128
NWIN_AGG = EDGES_PER_W // K_AGG
K_NRM = 1296
NWIN_NRM = EDGES_PER_W // K_NRM
EDGES_PER_T = E_TOT // NS
K_DEG = 2592
NWIN_DEG = EDGES_PER_T // K_DEG

_MESH = plsc.VectorSubcoreMesh(core_axis_name="c", subcore_axis_name="s")


def _newton_rsqrt(x):
    xb = lax.bitcast_convert_type(x, jnp.int32)
    y = lax.bitcast_convert_type(jnp.int32(0x5F3759DF) - (xb >> 1),
                                 jnp.float32)
    for _ in range(4):
        y = y * (1.5 - 0.5 * x * y * y)
    return y


def _sc_norm_body(src_hbm, dst_hbm, ew_hbm, batch_hbm, zeros_hbm,
                  norm_hbm, qpart_hbm,
                  deg_acc, q_acc, dis_sh,
                  dstb_a, ewb_a, dbuf, disb, dis_full, batch_full,
                  srcb, dstb, ewb, normb, qidxb):
    cid = lax.axis_index("c")
    tid = lax.axis_index("s")
    wid = tid * NC + cid

    @pl.when(tid == 0)
    def _():
        pltpu.sync_copy(zeros_hbm.at[pl.ds(0, NP)], deg_acc)
        pltpu.sync_copy(zeros_hbm, q_acc)

    plsc.subcore_barrier()

    def deg_win(w, carry):
        base = pl.multiple_of(tid * EDGES_PER_T + w * K_DEG, 8)
        pltpu.sync_copy(dst_hbm.at[pl.ds(base, K_DEG)], dstb_a)
        pltpu.sync_copy(ew_hbm.at[pl.ds(base, K_DEG)], ewb_a)
        pltpu.sync_copy(ewb_a, deg_acc.at[dstb_a], add=True)
        return carry

    lax.fori_loop(0, NWIN_DEG, deg_win, 0)
    plsc.subcore_barrier()

    rb = tid * (NP // NS)
    pltpu.sync_copy(deg_acc.at[pl.ds(rb, NP // NS)], dbuf)
    for i in range(NP // NS // 16):
        sl = pl.ds(i * 16, 16)
        disb[sl] = _newton_rsqrt(dbuf[sl])
    pltpu.sync_copy(disb, dis_sh.at[pl.ds(rb, NP // NS)])
    plsc.subcore_barrier()
    pltpu.sync_copy(dis_sh, dis_full)
    pltpu.sync_copy(batch_hbm, batch_full.at[pl.ds(0, N)])

    def nrm_win(w, carry):
        base = pl.multiple_of(wid * EDGES_PER_W + w * K_NRM, 8)
        pltpu.sync_copy(src_hbm.at[pl.ds(base, K_NRM)], srcb)
        pltpu.sync_copy(dst_hbm.at[pl.ds(base, K_NRM)], dstb)
        pltpu.sync_copy(ew_hbm.at[pl.ds(base, K_NRM)], ewb)
        for g in range(K_NRM // 16):
            sl = pl.ds(g * 16, 16)
            s16 = srcb[sl]
            d16 = dstb[sl]
            gs = plsc.load_gather(dis_full, [s16])
            gd = plsc.load_gather(dis_full, [d16])
            normb[sl] = gs * ewb[sl] * gd
            bg = plsc.load_gather(batch_full, [d16])
            qidxb[sl] = s16 * NG + bg
        pltpu.sync_copy(normb, norm_hbm.at[pl.ds(base, K_NRM)])
        pltpu.sync_copy(normb, q_acc.at[qidxb], add=True)
        return carry

    lax.fori_loop(0, NWIN_NRM, nrm_win, 0)
    plsc.subcore_barrier()

    qb = pl.multiple_of(tid * 5120, 128)

    @pl.when(tid < NS - 1)
    def _():
        pltpu.sync_copy(q_acc.at[pl.ds(qb, 5120)],
                        qpart_hbm.at[cid, 0, pl.ds(qb, 5120)])

    @pl.when(tid == NS - 1)
    def _():
        pltpu.sync_copy(q_acc.at[pl.ds(qb, 3200)],
                        qpart_hbm.at[cid, 0, pl.ds(qb, 3200)])


_sc_norm = pl.kernel(
    _sc_norm_body,
    out_type=(
        jax.ShapeDtypeStruct((E_TOT,), jnp.float32),
        jax.ShapeDtypeStruct((NC, 1, NG * N), jnp.float32),
    ),
    mesh=_MESH,
    scratch_types=[
        pltpu.VMEM_SHARED((NP,), jnp.float32),
        pltpu.VMEM_SHARED((NG * N,), jnp.float32),
        pltpu.VMEM_SHARED((NP,), jnp.float32),
        pltpu.VMEM((K_DEG,), jnp.int32),
        pltpu.VMEM((K_DEG,), jnp.float32),
        pltpu.VMEM((NP // NS,), jnp.float32),
        pltpu.VMEM((NP // NS,), jnp.float32),
        pltpu.VMEM((NP,), jnp.float32),
        pltpu.VMEM((NP,), jnp.int32),
        pltpu.VMEM((K_NRM,), jnp.int32),
        pltpu.VMEM((K_NRM,), jnp.int32),
        pltpu.VMEM((K_NRM,), jnp.float32),
        pltpu.VMEM((K_NRM,), jnp.float32),
        pltpu.VMEM((K_NRM,), jnp.int32),
    ],
    compiler_params=pltpu.CompilerParams(needs_layout_passes=False),
    name="sc_gcn_norm",
)


def _sc_agg_body(h_hbm, ed_hbm, dst_hbm, zeros_hbm,
                 parts_hbm,
                 acc, ep0, dstb0, rows0, ep1, dstb1, rows1,
                 semi, semg, sems):
    cid = lax.axis_index("c")
    tid = lax.axis_index("s")
    wid = tid * NC + cid

    @pl.when(tid == 0)
    def _():
        pltpu.sync_copy(zeros_hbm, acc)

    plsc.subcore_barrier()

    eb = wid * EDGES_PER_W

    def stage(w, ep, db):
        base = pl.multiple_of(eb + w * K_AGG, 128)
        pltpu.async_copy(ed_hbm.at[:, pl.ds(base, K_AGG)], ep, semi)
        pltpu.async_copy(dst_hbm.at[pl.ds(base, K_AGG)], db, semi)

    pltpu.sync_copy(ed_hbm.at[:, pl.ds(eb, K_AGG)], ep0)
    pltpu.sync_copy(dst_hbm.at[pl.ds(eb, K_AGG)], dstb0)
    pltpu.async_copy(h_hbm.at[ep0.at[0]], rows0, semg)

    def process(w, ep, db, rw, epn, dbn, rn, have_next):
        nxt = w + 1

        @pl.when(w >= 1)
        def _():
            pltpu.make_async_copy(rn, acc.at[dbn], sems).wait()

        @pl.when(have_next)
        def _():
            stage(nxt, epn, dbn)

        pltpu.make_async_copy(h_hbm.at[ep.at[0]], rw, semg).wait()

        @pl.when(have_next)
        def _():
            pltpu.make_async_copy(
                ed_hbm.at[:, pl.ds(0, K_AGG)], epn, semi).wait()
            pltpu.make_async_copy(
                dst_hbm.at[pl.ds(0, K_AGG)], dbn, semi).wait()
            pltpu.async_copy(h_hbm.at[epn.at[0]], rn, semg)

        @plsc.parallel_loop(0, K_AGG, step=1, unroll=8)
        def _(e):
            nv = lax.bitcast_convert_type(
                plsc.load_gather(ep.at[1],
                                 [jnp.zeros((16,), jnp.int32) + e]),
                jnp.float32)
            for c in range(DH // 16):
                sl = pl.ds(c * 16, 16)
                rw[e, sl] = rw[e, sl] * nv

        pltpu.async_copy(rw, acc.at[db], sems, add=True)

    def win_pair(p, carry):
        w0 = p * 2
        process(w0, ep0, dstb0, rows0, ep1, dstb1, rows1,
                w0 + 1 < NWIN_AGG)
        process(w0 + 1, ep1, dstb1, rows1, ep0, dstb0, rows0,
                w0 + 2 < NWIN_AGG)
        return carry

    lax.fori_loop(0, NWIN_AGG // 2, win_pair, 0)
    process(NWIN_AGG - 1, ep0, dstb0, rows0, ep1, dstb1, rows1, False)
    pltpu.make_async_copy(rows0, acc.at[dstb0], sems).wait()
    plsc.subcore_barrier()

    rb = pl.multiple_of(tid * 624, 8)

    @pl.when(tid < NS - 1)
    def _():
        pltpu.sync_copy(acc.at[pl.ds(rb, 624)],
                        parts_hbm.at[cid, pl.ds(rb, 624)])

    @pl.when(tid == NS - 1)
    def _():
        pltpu.sync_copy(acc.at[pl.ds(rb, 640)],
                        parts_hbm.at[cid, pl.ds(rb, 640)])


_sc_agg = pl.kernel(
    _sc_agg_body,
    out_type=jax.ShapeDtypeStruct((NC, N, DH), jnp.float32),
    mesh=_MESH,
    scratch_types=[
        pltpu.VMEM_SHARED((N, DH), jnp.float32),
        pltpu.VMEM((2, K_AGG), jnp.int32),
        pltpu.VMEM((K_AGG,), jnp.int32),
        pltpu.VMEM((K_AGG, DH), jnp.float32),
        pltpu.VMEM((2, K_AGG), jnp.int32),
        pltpu.VMEM((K_AGG,), jnp.int32),
        pltpu.VMEM((K_AGG, DH), jnp.float32),
        pltpu.SemaphoreType.DMA,
        pltpu.SemaphoreType.DMA,
        pltpu.SemaphoreType.DMA,
    ],
    compiler_params=pltpu.CompilerParams(needs_layout_passes=False),
    name="sc_gcn_agg",
)


RB = 1000
NB = N // RB


def _mm_body(x_ref, w_ref, o_ref):
    o_ref[...] = jnp.dot(x_ref[...], w_ref[...],
                         preferred_element_type=jnp.float32)


_tc_mm = pl.pallas_call(
    _mm_body,
    grid=(NB,),
    in_specs=[
        pl.BlockSpec((RB, DH), lambda i: (i, 0)),
        pl.BlockSpec((DH, DH), lambda i: (0, 0)),
    ],
    out_specs=pl.BlockSpec((RB, DH), lambda i: (i, 0)),
    out_shape=jax.ShapeDtypeStruct((N, DH), jnp.float32),
)


def _relu_mm_body(p_ref, b_ref, w_ref, o_ref):
    h = jnp.maximum(p_ref[0] + p_ref[1] + b_ref[...], 0.0)
    o_ref[...] = jnp.dot(h, w_ref[...], preferred_element_type=jnp.float32)


_tc_relu_mm = pl.pallas_call(
    _relu_mm_body,
    grid=(NB,),
    in_specs=[
        pl.BlockSpec((NC, RB, DH), lambda i: (0, i, 0)),
        pl.BlockSpec((1, DH), lambda i: (0, 0)),
        pl.BlockSpec((DH, DH), lambda i: (0, 0)),
    ],
    out_specs=pl.BlockSpec((RB, DH), lambda i: (i, 0)),
    out_shape=jax.ShapeDtypeStruct((N, DH), jnp.float32),
)


def _head_body(p_ref, b2_ref, qp_ref, batch_ref, w3_ref, b3_ref,
               g_ref, be_ref, p1w_ref, p1b_ref, p2w_ref, p2b_ref,
               o_ref, z_acc, c_acc):
    i = pl.program_id(0)

    @pl.when(i == 0)
    def _():
        z_acc[...] = jnp.zeros_like(z_acc)
        c_acc[...] = jnp.zeros_like(c_acc)

    h2 = jnp.maximum(p_ref[0] + p_ref[1] + b2_ref[...], 0.0)
    hm = jnp.dot(h2, w3_ref[...],
                 preferred_element_type=jnp.float32)
    qs = qp_ref[0] + qp_ref[1]
    dn = (((0,), (0,)), ((), ()))
    z_acc[...] += lax.dot_general(qs, hm, dimension_numbers=dn,
                                  preferred_element_type=jnp.float32,
                                  precision=lax.Precision.HIGHEST)
    m = (batch_ref[...] ==
         lax.broadcasted_iota(jnp.int32, (RB, NG), 1)).astype(jnp.float32)
    cnt = lax.dot_general(m, jnp.ones((RB, 1), jnp.float32),
                          dimension_numbers=dn,
                          preferred_element_type=jnp.float32,
                          precision=lax.Precision.HIGHEST)
    c_acc[...] += jnp.broadcast_to(cnt, (NG, 200))

    @pl.when(i == NB - 1)
    def _():
        counts = jnp.maximum(c_acc[...], 1.0)
        y = z_acc[...] / counts + b3_ref[...]
        mu = jnp.mean(y, axis=-1, keepdims=True)
        var = jnp.mean((y - mu) ** 2, axis=-1, keepdims=True)
        yn = (y - mu) * lax.rsqrt(var + 1e-5) * g_ref[...] + be_ref[...]
        t = jnp.maximum(
            jnp.dot(yn, p1w_ref[...],
                    preferred_element_type=jnp.float32) + p1b_ref[...], 0.0)
        o_ref[...] = jnp.dot(t, p2w_ref[...],
                             preferred_element_type=jnp.float32) + p2b_ref[...]


_tc_head = pl.pallas_call(
    _head_body,
    grid=(NB,),
    in_specs=[
        pl.BlockSpec((NC, RB, DH), lambda i: (0, i, 0)),
        pl.BlockSpec((1, DH), lambda i: (0, 0)),
        pl.BlockSpec((NC, RB, NG), lambda i: (0, i, 0)),
        pl.BlockSpec((RB, 1), lambda i: (i, 0)),
        pl.BlockSpec((DH, 200), lambda i: (0, 0)),
        pl.BlockSpec((1, 200), lambda i: (0, 0)),
        pl.BlockSpec((1, 200), lambda i: (0, 0)),
        pl.BlockSpec((1, 200), lambda i: (0, 0)),
        pl.BlockSpec((200, DH), lambda i: (0, 0)),
        pl.BlockSpec((1, DH), lambda i: (0, 0)),
        pl.BlockSpec((DH, 4), lambda i: (0, 0)),
        pl.BlockSpec((1, 4), lambda i: (0, 0)),
    ],
    out_specs=pl.BlockSpec((NG, 4), lambda i: (0, 0)),
    out_shape=jax.ShapeDtypeStruct((NG, 4), jnp.float32),
    scratch_shapes=[
        pltpu.VMEM((NG, 200), jnp.float32),
        pltpu.VMEM((NG, 200), jnp.float32),
    ],
)


def kernel(x, edge_index, edge_attr, batch, W1, b1, W2, b2, W3, b3,
           ln_gamma, ln_beta, P1_W, P1_b, P2_W, P2_b):
    src = edge_index[0]
    dst = edge_index[1]
    iota = jnp.arange(N, dtype=jnp.int32)
    padn = E_TOT - E - N
    zpad_i = jnp.zeros((padn,), jnp.int32)
    src_e = jnp.concatenate([src, iota, zpad_i])
    dst_e = jnp.concatenate([dst, iota, zpad_i])
    ew_e = jnp.concatenate([edge_attr, jnp.ones((N,), jnp.float32),
                            jnp.zeros((padn,), jnp.float32)])
    zeros_q = jnp.zeros((NG * N,), jnp.float32)
    zeros_h = jnp.zeros((N, DH), jnp.float32)

    norm, qpart = _sc_norm(src_e, dst_e, ew_e, batch, zeros_q)
    ed = jnp.stack([src_e, lax.bitcast_convert_type(norm, jnp.int32)])

    h1a = _tc_mm(x, W1)
    p1 = _sc_agg(h1a, ed, dst_e, zeros_h)
    h2a = _tc_relu_mm(p1, b1.reshape(1, DH), W2)
    p2 = _sc_agg(h2a, ed, dst_e, zeros_h)

    return _tc_head(
        p2, b2.reshape(1, DH), qpart.reshape(NC, N, NG),
        batch.reshape(N, 1), W3, b3.reshape(1, 200),
        ln_gamma.reshape(1, 200), ln_beta.reshape(1, 200),
        P1_W, P1_b.reshape(1, DH), P2_W, P2_b.reshape(1, 4))
